# baseline (speedup 1.0000x reference)
import jax
import jax.numpy as jnp
from jax import lax
from jax.experimental import pallas as pl
from jax.experimental.pallas import tpu as pltpu
from jax.experimental.pallas import tpu_sc as plsc

NC, NS, L = 2, 16, 16
NW = NC * NS
B, S, D = 4, 2048, 128
T = B * S
TPW = T // NW
CHUNKS = (32, 96, 96, 32)
OFFS = (0, 32, 128, 224)
NCH = len(CHUNKS)


def _body(ids_hbm, tok_hbm, pos_hbm, out_hbm, idx_v, rows_v,
          s_idx, s_p0, s_p1, s_p2, s_p3, s_g0, s_g1, s_g2, s_g3, s_out):
    wid = lax.axis_index("s") * NC + lax.axis_index("c")
    base = wid * TPW
    pos_base = lax.rem(base, S)
    s_p = (s_p0, s_p1, s_p2, s_p3)
    s_g = (s_g0, s_g1, s_g2, s_g3)

    c_idx = pltpu.async_copy(ids_hbm.at[pl.ds(base, TPW)], idx_v, s_idx)
    c_pos = [
        pltpu.async_copy(
            pos_hbm.at[pl.ds(pos_base + OFFS[j], CHUNKS[j])],
            rows_v.at[pl.ds(OFFS[j], CHUNKS[j])],
            s_p[j],
        )
        for j in range(NCH)
    ]
    c_idx.wait()
    gathers = []
    for j in range(NCH):
        c_pos[j].wait()
        gathers.append(
            pltpu.async_copy(
                tok_hbm.at[idx_v.at[pl.ds(OFFS[j], CHUNKS[j])]],
                rows_v.at[pl.ds(OFFS[j], CHUNKS[j])],
                s_g[j],
                add=True,
            )
        )
    outs = []
    for j in range(NCH):
        gathers[j].wait()
        outs.append(
            pltpu.async_copy(
                rows_v.at[pl.ds(OFFS[j], CHUNKS[j])],
                out_hbm.at[pl.ds(base + OFFS[j], CHUNKS[j])],
                s_out,
            )
        )
    for c in outs:
        c.wait()


@jax.jit
def _embed(ids_flat, tok, pos):
    mesh = plsc.VectorSubcoreMesh(
        core_axis_name="c", subcore_axis_name="s", num_cores=NC, num_subcores=NS
    )
    return pl.kernel(
        _body,
        out_type=jax.ShapeDtypeStruct((T, D), jnp.float32),
        mesh=mesh,
        scratch_types=[
            pltpu.VMEM((TPW,), jnp.int32),
            pltpu.VMEM((TPW, D), jnp.float32),
            pltpu.SemaphoreType.DMA,
            pltpu.SemaphoreType.DMA,
            pltpu.SemaphoreType.DMA,
            pltpu.SemaphoreType.DMA,
            pltpu.SemaphoreType.DMA,
            pltpu.SemaphoreType.DMA,
            pltpu.SemaphoreType.DMA,
            pltpu.SemaphoreType.DMA,
            pltpu.SemaphoreType.DMA,
            pltpu.SemaphoreType.DMA,
        ],
    )(ids_flat, tok, pos)


def kernel(input_ids, token_table, pos_table):
    ids_flat = input_ids.reshape(T).astype(jnp.int32)
    out = _embed(ids_flat, token_table, pos_table)
    return out.reshape(B, S, D)


# speedup vs baseline: 1.0101x; 1.0101x over previous
import jax
import jax.numpy as jnp
from jax import lax
from jax.experimental import pallas as pl
from jax.experimental.pallas import tpu as pltpu
from jax.experimental.pallas import tpu_sc as plsc

NC, NS, L = 2, 16, 16
NW = NC * NS
B, S, D = 4, 2048, 128
T = B * S
TPW = T // NW
CH = 128
NCH = TPW // CH


def _body(ids_hbm, tok_hbm, pos_hbm, out_hbm, idx_v, rows_v,
          s_idx, s_p0, s_p1, s_g0, s_g1, s_out):
    wid = lax.axis_index("s") * NC + lax.axis_index("c")
    base = wid * TPW
    pos_base = lax.rem(base, S)
    s_p = (s_p0, s_p1)
    s_g = (s_g0, s_g1)

    c_idx = pltpu.async_copy(ids_hbm.at[pl.ds(base, TPW)], idx_v, s_idx)
    c_pos = [
        pltpu.async_copy(
            pos_hbm.at[pl.ds(pos_base + j * CH, CH)],
            rows_v.at[pl.ds(j * CH, CH)],
            s_p[j],
        )
        for j in range(NCH)
    ]
    c_idx.wait()
    gathers = []
    for j in range(NCH):
        c_pos[j].wait()
        gathers.append(
            pltpu.async_copy(
                tok_hbm.at[idx_v.at[pl.ds(j * CH, CH)]],
                rows_v.at[pl.ds(j * CH, CH)],
                s_g[j],
                add=True,
            )
        )
    outs = []
    for j in range(NCH):
        gathers[j].wait()
        outs.append(
            pltpu.async_copy(
                rows_v.at[pl.ds(j * CH, CH)],
                out_hbm.at[pl.ds(base + j * CH, CH)],
                s_out,
            )
        )
    for c in outs:
        c.wait()


@jax.jit
def _embed(ids_flat, tok, pos):
    mesh = plsc.VectorSubcoreMesh(
        core_axis_name="c", subcore_axis_name="s", num_cores=NC, num_subcores=NS
    )
    return pl.kernel(
        _body,
        out_type=jax.ShapeDtypeStruct((T, D), jnp.float32),
        mesh=mesh,
        scratch_types=[
            pltpu.VMEM((TPW,), jnp.int32),
            pltpu.VMEM((TPW, D), jnp.float32),
            pltpu.SemaphoreType.DMA,
            pltpu.SemaphoreType.DMA,
            pltpu.SemaphoreType.DMA,
            pltpu.SemaphoreType.DMA,
            pltpu.SemaphoreType.DMA,
            pltpu.SemaphoreType.DMA,
        ],
    )(ids_flat, tok, pos)


def kernel(input_ids, token_table, pos_table):
    ids_flat = input_ids.reshape(T).astype(jnp.int32)
    out = _embed(ids_flat, token_table, pos_table)
    return out.reshape(B, S, D)
